# baseline (speedup 1.0000x reference)
import functools

import jax
import jax.numpy as jnp
from jax import lax
from jax.experimental import pallas as pl
from jax.experimental.pallas import tpu as pltpu
from jax.experimental.pallas import tpu_sc as plsc

SEQ = 512
CLS_ID = 101
SEP_ID = 102
EMB_D = 128
ZBASE = 30522
VOCAB_PAD = 31040
NC = 2
NS = 16
NW = NC * NS
K = 64
CPR = SEQ // K
ROWS_W = 32
VPR = SEQ // 16


def _pack_body(body_ref, len_ref, packed_ref, mask_ref):
    bm = body_ref.shape[0]
    pos = lax.broadcasted_iota(jnp.int32, (bm, SEQ), 1)
    L = len_ref[...]
    body = body_ref[...]
    packed = jnp.where(pos == 0, CLS_ID,
             jnp.where(pos <= L, body,
             jnp.where(pos == L + 1, SEP_ID, 0)))
    mask = pos <= L + 1
    packed_ref[...] = packed
    mask_ref[...] = mask


def _pack_call(body, lengths2d):
    B = body.shape[0]
    bm = 256
    grid = B // bm
    return pl.pallas_call(
        _pack_body,
        grid=(grid,),
        in_specs=[pl.BlockSpec((bm, SEQ), lambda i: (i, 0)),
                  pl.BlockSpec((bm, 1), lambda i: (i, 0))],
        out_specs=[pl.BlockSpec((bm, SEQ), lambda i: (i, 0))] * 2,
        out_shape=[jax.ShapeDtypeStruct((B, SEQ), jnp.int32),
                   jax.ShapeDtypeStruct((B, SEQ), jnp.bool_)],
    )(body, lengths2d)


def _sc_gather(body_flat, len16, table_pad):
    BT = body_flat.shape[0]
    span = BT // NW
    R = ROWS_W
    mesh = plsc.VectorSubcoreMesh(core_axis_name="c", subcore_axis_name="s")

    @functools.partial(
        pl.kernel, mesh=mesh,
        out_type=jax.ShapeDtypeStruct((BT, EMB_D), jnp.float32),
        scratch_types=(
            [pltpu.VMEM((span,), jnp.int32),
             pltpu.VMEM((span,), jnp.int32),
             pltpu.VMEM((ROWS_W * 16,), jnp.int32),
             pltpu.VMEM((K, EMB_D), jnp.float32)]
            + [pltpu.VMEM((K, EMB_D), jnp.float32) for _ in range(CPR)]
            + [pltpu.SemaphoreType.DMA for _ in range(2 * CPR)]
        ),
    )
    def k(body_hbm, len_hbm, table_hbm, out_hbm,
          body_v, idx_v, len_v, zbuf, *rest):
        bufs = rest[:CPR]
        gsem = rest[CPR:2 * CPR]
        ssem = rest[2 * CPR:3 * CPR]
        wid = lax.axis_index("s") * NC + lax.axis_index("c")
        base = wid * span
        pltpu.sync_copy(body_hbm.at[pl.ds(base, span)], body_v)
        pltpu.sync_copy(len_hbm.at[pl.ds(wid * (ROWS_W * 16), ROWS_W * 16)],
                        len_v)
        pltpu.sync_copy(table_hbm.at[pl.ds(30528, K)], zbuf)

        lane = lax.iota(jnp.int32, 16)

        def len_of_row(i):
            return len_v[pl.ds(pl.multiple_of(i * 16, 16), 16)][0]

        def row_preds(L):
            n = (L + 1) // K + 1
            return [n > j for j in range(CPR)]

        VPC = K // 16

        def compute_idx(i, L, preds):
            rb = pl.multiple_of(i * SEQ, 16)
            for j in range(CPR):
                @pl.when(preds[j])
                def _():
                    for cv in range(j * VPC, (j + 1) * VPC):
                        pos = lane + (cv * 16)
                        body = body_v[pl.ds(rb + cv * 16, 16)]
                        v = jnp.where(pos == 0, CLS_ID,
                            jnp.where(pos <= L, body,
                            jnp.where(pos == L + 1, SEP_ID, ZBASE + pos)))
                        idx_v[pl.ds(rb + cv * 16, 16)] = v

        def g_start(c, j):
            pltpu.async_copy(table_hbm.at[idx_v.at[pl.ds(c * K, K)]],
                             bufs[j], gsem[j])

        def g_wait(j):
            pltpu.make_async_copy(table_hbm.at[idx_v.at[pl.ds(0, K)]],
                                  bufs[j], gsem[j]).wait()

        def s_start(c, j, src):
            pltpu.async_copy(src, out_hbm.at[pl.ds(base + c * K, K)],
                             ssem[j])

        def s_wait(j):
            pltpu.make_async_copy(bufs[j], out_hbm.at[pl.ds(0, K)],
                                  ssem[j]).wait()

        L0 = len_of_row(0)
        preds0 = row_preds(L0)
        compute_idx(0, L0, preds0)
        for j in range(CPR):
            @pl.when(preds0[j])
            def _():
                g_start(j, j)

        def outer(i, carry):
            L_i = len_of_row(i)
            preds = row_preds(L_i)
            cb = i * CPR
            for j in range(CPR):
                gathered = preds[j]

                @pl.when(gathered)
                def _():
                    g_wait(j)
                    s_start(cb + j, j, bufs[j])

                @pl.when(jnp.logical_not(gathered))
                def _():
                    s_start(cb + j, j, zbuf)

            @pl.when(i + 1 < R)
            def _():
                L_n = len_of_row(i + 1)
                preds_n = row_preds(L_n)
                compute_idx(i + 1, L_n, preds_n)
                for j in range(CPR):
                    s_wait(j)

                    @pl.when(preds_n[j])
                    def _():
                        g_start(cb + CPR + j, j)

            return carry

        lax.fori_loop(0, R, outer, 0)
        for j in range(CPR):
            s_wait(j)

    return k(body_flat, len16, table_pad)


def kernel(token_ids, lengths, table):
    B = token_ids.shape[0]
    body = jnp.pad(token_ids, ((0, 0), (1, 1)))
    len16 = jnp.broadcast_to(lengths[:, None], (B, 16))
    table_pad = jnp.pad(table, ((0, VOCAB_PAD - table.shape[0]), (0, 0)))
    emb = _sc_gather(body.reshape(-1), len16.reshape(-1),
                     table_pad).reshape(B, SEQ, EMB_D)
    packed, mask = _pack_call(body, lengths[:, None])
    segment_ids = jnp.zeros((B, SEQ), jnp.int32)
    return packed, segment_ids, mask, emb


# speedup vs baseline: 1.0041x; 1.0041x over previous
import functools

import jax
import jax.numpy as jnp
from jax import lax
from jax.experimental import pallas as pl
from jax.experimental.pallas import tpu as pltpu
from jax.experimental.pallas import tpu_sc as plsc

SEQ = 512
CLS_ID = 101
SEP_ID = 102
EMB_D = 128
NC = 2
NS = 16
NW = NC * NS
K = 64
CPR = SEQ // K
ROWS_W = 32
VPR = SEQ // 16


def _pack_body(body_ref, len_ref, packed_ref, mask_ref):
    bm = body_ref.shape[0]
    pos = lax.broadcasted_iota(jnp.int32, (bm, SEQ), 1)
    L = len_ref[...]
    body = body_ref[...]
    packed = jnp.where(pos == 0, CLS_ID,
             jnp.where(pos <= L, body,
             jnp.where(pos == L + 1, SEP_ID, 0)))
    mask = pos <= L + 1
    packed_ref[...] = packed
    mask_ref[...] = mask


def _pack_call(body, lengths2d):
    B = body.shape[0]
    bm = 256
    grid = B // bm
    return pl.pallas_call(
        _pack_body,
        grid=(grid,),
        in_specs=[pl.BlockSpec((bm, SEQ), lambda i: (i, 0)),
                  pl.BlockSpec((bm, 1), lambda i: (i, 0))],
        out_specs=[pl.BlockSpec((bm, SEQ), lambda i: (i, 0))] * 2,
        out_shape=[jax.ShapeDtypeStruct((B, SEQ), jnp.int32),
                   jax.ShapeDtypeStruct((B, SEQ), jnp.bool_)],
    )(body, lengths2d)


def _sc_gather(body_flat, len16, table):
    BT = body_flat.shape[0]
    span = BT // NW
    R = ROWS_W
    mesh = plsc.VectorSubcoreMesh(core_axis_name="c", subcore_axis_name="s")

    @functools.partial(
        pl.kernel, mesh=mesh,
        out_type=jax.ShapeDtypeStruct((BT, EMB_D), jnp.float32),
        scratch_types=(
            [pltpu.VMEM((span,), jnp.int32),
             pltpu.VMEM((span,), jnp.int32),
             pltpu.VMEM((ROWS_W * 16,), jnp.int32),
             pltpu.VMEM((K, EMB_D), jnp.float32)]
            + [pltpu.VMEM((K, EMB_D), jnp.float32) for _ in range(CPR)]
            + [pltpu.SemaphoreType.DMA for _ in range(2 * CPR)]
        ),
    )
    def k(body_hbm, len_hbm, table_hbm, out_hbm,
          body_v, idx_v, len_v, zbuf, *rest):
        bufs = rest[:CPR]
        gsem = rest[CPR:2 * CPR]
        ssem = rest[2 * CPR:3 * CPR]
        wid = lax.axis_index("s") * NC + lax.axis_index("c")
        base = wid * span
        pltpu.sync_copy(body_hbm.at[pl.ds(base, span)], body_v)
        pltpu.sync_copy(len_hbm.at[pl.ds(wid * (ROWS_W * 16), ROWS_W * 16)],
                        len_v)
        lane = lax.iota(jnp.int32, 16)
        zvec = jnp.zeros((16,), jnp.float32)
        for r in range(K):
            for c in range(EMB_D // 16):
                zbuf[r, pl.ds(c * 16, 16)] = zvec

        def len_of_row(i):
            return len_v[pl.ds(pl.multiple_of(i * 16, 16), 16)][0]

        def row_nch(L):
            return (L + 1) // K + 1

        VPC = K // 16

        def compute_idx(i, L, preds):
            rb = pl.multiple_of(i * SEQ, 16)
            for j in range(CPR):
                @pl.when(preds[j])
                def _():
                    for cv in range(j * VPC, (j + 1) * VPC):
                        pos = lane + (cv * 16)
                        body = body_v[pl.ds(rb + cv * 16, 16)]
                        v = jnp.where(pos == 0, CLS_ID,
                            jnp.where(pos <= L, body,
                            jnp.where(pos == L + 1, SEP_ID, pos)))
                        idx_v[pl.ds(rb + cv * 16, 16)] = v

        def g_start(c, j):
            pltpu.async_copy(table_hbm.at[idx_v.at[pl.ds(c * K, K)]],
                             bufs[j], gsem[j])

        def g_wait(j):
            pltpu.make_async_copy(table_hbm.at[idx_v.at[pl.ds(0, K)]],
                                  bufs[j], gsem[j]).wait()

        def s_start(c, j, src):
            pltpu.async_copy(src, out_hbm.at[pl.ds(base + c * K, K)],
                             ssem[j])

        def s_wait(j):
            pltpu.make_async_copy(bufs[j], out_hbm.at[pl.ds(0, K)],
                                  ssem[j]).wait()

        L0 = len_of_row(0)
        n0 = row_nch(L0)
        preds0 = [n0 > j for j in range(CPR)]
        compute_idx(0, L0, preds0)
        for j in range(CPR):
            @pl.when(preds0[j])
            def _():
                g_start(j, j)

        def outer(i, carry):
            L_i = len_of_row(i)
            n_i = row_nch(L_i)
            preds = [n_i > j for j in range(CPR)]
            cb = i * CPR
            for j in range(CPR):
                gathered = preds[j]

                @pl.when(gathered)
                def _():
                    g_wait(j)

                    @pl.when(j == n_i - 1)
                    def _():
                        t = (L_i + 2) - j * K

                        def zrow(r, carry):
                            for c in range(EMB_D // 16):
                                bufs[j][r, pl.ds(c * 16, 16)] = zvec
                            return carry

                        lax.fori_loop(t, K, zrow, 0)

                    s_start(cb + j, j, bufs[j])

                @pl.when(jnp.logical_not(gathered))
                def _():
                    s_start(cb + j, j, zbuf)

            @pl.when(i + 1 < R)
            def _():
                L_n = len_of_row(i + 1)
                n_n = row_nch(L_n)
                preds_n = [n_n > j for j in range(CPR)]
                compute_idx(i + 1, L_n, preds_n)
                for j in range(CPR):
                    s_wait(j)

                    @pl.when(preds_n[j])
                    def _():
                        g_start(cb + CPR + j, j)

            return carry

        lax.fori_loop(0, R, outer, 0)
        for j in range(CPR):
            s_wait(j)

    return k(body_flat, len16, table)


def kernel(token_ids, lengths, table):
    B = token_ids.shape[0]
    body = jnp.pad(token_ids, ((0, 0), (1, 1)))
    len16 = jnp.broadcast_to(lengths[:, None], (B, 16))
    emb = _sc_gather(body.reshape(-1), len16.reshape(-1),
                     table).reshape(B, SEQ, EMB_D)
    packed, mask = _pack_call(body, lengths[:, None])
    segment_ids = jnp.zeros((B, SEQ), jnp.int32)
    return packed, segment_ids, mask, emb
